# baseline (speedup 1.0000x reference)
import functools

import jax
import jax.numpy as jnp
from jax import lax
from jax.experimental import pallas as pl
from jax.experimental.pallas import tpu as pltpu
from jax.experimental.pallas import tpu_sc as plsc

_NC = 2
_NS = 16
_NW = _NC * _NS
_L = 16

_R = 4
_NBUF = 2


def _sc_permute(zf, perm_i32):
    n_elems, = zf.shape
    d = 4096
    n_rows = n_elems // d
    rows_per_tile = n_rows // _NW
    groups = rows_per_tile // _R
    n_chunks = d // _L

    mesh = plsc.VectorSubcoreMesh(core_axis_name="c", subcore_axis_name="s")

    @functools.partial(
        pl.kernel,
        out_type=jax.ShapeDtypeStruct((n_elems,), jnp.float32),
        mesh=mesh,
        scratch_types=[
            pltpu.VMEM((d,), jnp.int32),
            [pltpu.VMEM((_R * d,), jnp.float32) for _ in range(_NBUF)],
            [pltpu.VMEM((_R * d,), jnp.float32) for _ in range(_NBUF)],
            [pltpu.SemaphoreType.DMA for _ in range(_NBUF)],
            [pltpu.SemaphoreType.DMA for _ in range(_NBUF)],
        ],
        compiler_params=pltpu.CompilerParams(needs_layout_passes=False),
    )
    def permute_kernel(z_hbm, perm_hbm, out_hbm, perm_v, inbufs, outbufs,
                       in_sems, out_sems):
        wid = lax.axis_index("s") * _NC + lax.axis_index("c")
        base = wid * (rows_per_tile * d)

        pltpu.sync_copy(perm_hbm, perm_v)

        def start_in(g, b):
            pltpu.async_copy(
                z_hbm.at[pl.ds(base + g * (_R * d), _R * d)], inbufs[b],
                in_sems[b])

        def wait_in(b):
            pltpu.make_async_copy(
                z_hbm.at[pl.ds(0, _R * d)], inbufs[b], in_sems[b]).wait()

        def start_out(g, b):
            pltpu.async_copy(
                outbufs[b], out_hbm.at[pl.ds(base + g * (_R * d), _R * d)],
                out_sems[b])

        def wait_out(b):
            pltpu.make_async_copy(
                outbufs[b], out_hbm.at[pl.ds(0, _R * d)], out_sems[b]).wait()

        def permute_group(inbuf, outbuf):
            def jbody(j, carry):
                col = perm_v[pl.ds(j * _L, _L)]
                for r in range(_R):
                    outbuf[pl.ds(r * d + j * _L, _L)] = plsc.load_gather(
                        inbuf, [col + (r * d)])
                return carry
            lax.fori_loop(0, n_chunks, jbody, 0, unroll=2)

        for b in range(_NBUF):
            start_in(b, b)

        def gbody(gg, carry):
            for b in range(_NBUF):
                g = gg * _NBUF + b
                wait_in(b)

                @pl.when(gg > 0)
                def _():
                    wait_out(b)

                permute_group(inbufs[b], outbufs[b])
                start_out(g, b)

                @pl.when(g + _NBUF < groups)
                def _():
                    start_in(g + _NBUF, b)
            return carry

        lax.fori_loop(0, groups // _NBUF, gbody, 0)

        for b in range(_NBUF):
            wait_out(b)

    return permute_kernel(zf, perm_i32)


def kernel(z, perm):
    b, s, d = z.shape
    zf = z.reshape(b * s * d)
    out = _sc_permute(zf, perm.astype(jnp.int32))
    return out.reshape(b, s, d)


# speedup vs baseline: 1.8212x; 1.8212x over previous
import functools

import jax
import jax.numpy as jnp
from jax import lax
from jax.experimental import pallas as pl
from jax.experimental.pallas import tpu as pltpu
from jax.experimental.pallas import tpu_sc as plsc

_NC = 2
_NS = 16
_NW = _NC * _NS
_L = 16

_R = 4
_NBUF = 2


def _sc_permute(zf, perm_i32):
    n_elems, = zf.shape
    d = 4096
    n_rows = n_elems // d
    rows_per_tile = n_rows // _NW
    groups = rows_per_tile // _R
    n_chunks = d // _L

    mesh = plsc.VectorSubcoreMesh(core_axis_name="c", subcore_axis_name="s")

    @functools.partial(
        pl.kernel,
        out_type=jax.ShapeDtypeStruct((n_elems,), jnp.float32),
        mesh=mesh,
        scratch_types=[
            pltpu.VMEM((d,), jnp.int32),
            [pltpu.VMEM((_R * d,), jnp.float32) for _ in range(_NBUF)],
            [pltpu.VMEM((_R * d,), jnp.float32) for _ in range(_NBUF)],
            [pltpu.SemaphoreType.DMA for _ in range(_NBUF)],
            [pltpu.SemaphoreType.DMA for _ in range(_NBUF)],
        ],
        compiler_params=pltpu.CompilerParams(needs_layout_passes=False),
    )
    def permute_kernel(z_hbm, perm_hbm, out_hbm, perm_v, inbufs, outbufs,
                       in_sems, out_sems):
        wid = lax.axis_index("s") * _NC + lax.axis_index("c")
        base = wid * (rows_per_tile * d)

        pltpu.sync_copy(perm_hbm, perm_v)

        def start_in(g, b):
            pltpu.async_copy(
                z_hbm.at[pl.ds(base + g * (_R * d), _R * d)], inbufs[b],
                in_sems[b])

        def wait_in(b):
            pltpu.make_async_copy(
                z_hbm.at[pl.ds(0, _R * d)], inbufs[b], in_sems[b]).wait()

        def start_out(g, b):
            pltpu.async_copy(
                outbufs[b], out_hbm.at[pl.ds(base + g * (_R * d), _R * d)],
                out_sems[b])

        def wait_out(b):
            pltpu.make_async_copy(
                outbufs[b], out_hbm.at[pl.ds(0, _R * d)], out_sems[b]).wait()

        def permute_group(inbuf, outbuf):
            @plsc.parallel_loop(0, n_chunks, unroll=4)
            def _(j):
                col = perm_v[pl.ds(j * _L, _L)]
                for r in range(_R):
                    outbuf[pl.ds(r * d + j * _L, _L)] = plsc.load_gather(
                        inbuf, [col + (r * d)])

        for b in range(_NBUF):
            start_in(b, b)

        def gbody(gg, carry):
            for b in range(_NBUF):
                g = gg * _NBUF + b
                wait_in(b)

                @pl.when(gg > 0)
                def _():
                    wait_out(b)

                permute_group(inbufs[b], outbufs[b])
                start_out(g, b)

                @pl.when(g + _NBUF < groups)
                def _():
                    start_in(g + _NBUF, b)
            return carry

        lax.fori_loop(0, groups // _NBUF, gbody, 0)

        for b in range(_NBUF):
            wait_out(b)

    return permute_kernel(zf, perm_i32)


def kernel(z, perm):
    b, s, d = z.shape
    zf = z.reshape(b * s * d)
    out = _sc_permute(zf, perm.astype(jnp.int32))
    return out.reshape(b, s, d)
